# baseline (speedup 1.0000x reference)
import dataclasses
import functools

import jax
import jax.numpy as jnp
from jax import lax
from jax.experimental import pallas as pl
from jax.experimental.pallas import tpu as pltpu
from jax.experimental.pallas import tpu_sc as plsc

ALPHA = 0.01
K = 10

N = 10000
E = 320000
D = 128

NC = 2
NS = 16
L = 16
DH = D // NC

NP = 10240
G = 80
NCH = 250
RPT = NP // NS
RB = 80

_vector_mesh = plsc.VectorSubcoreMesh(core_axis_name="c", subcore_axis_name="s")

_sc_params = pltpu.CompilerParams()
for _f, _v in (("needs_layout_passes", False), ("use_tc_tiling_on_sc", False)):
    if _f in pltpu.CompilerParams.__dataclass_fields__:
        _sc_params = dataclasses.replace(_sc_params, **{_f: _v})


def _mlp_body(x_ref, w1_ref, b1_ref, w2_ref, b2_ref, o_ref):
    h = lax.dot_general(x_ref[...], w1_ref[...], (((1,), (1,)), ((), ())),
                        preferred_element_type=jnp.float32,
                        precision=lax.Precision.HIGHEST)
    h = jnp.maximum(h + b1_ref[...], 0.0)
    o = lax.dot_general(h, w2_ref[...], (((1,), (1,)), ((), ())),
                        preferred_element_type=jnp.float32,
                        precision=lax.Precision.HIGHEST)
    o_ref[...] = o + b2_ref[...]


def _mlp(x, W1, b1, W2, b2):
    bm = 2000
    return pl.pallas_call(
        _mlp_body,
        grid=(N // bm,),
        in_specs=[
            pl.BlockSpec((bm, D), lambda i: (i, 0)),
            pl.BlockSpec((D, D), lambda i: (0, 0)),
            pl.BlockSpec((1, D), lambda i: (0, 0)),
            pl.BlockSpec((D, D), lambda i: (0, 0)),
            pl.BlockSpec((1, D), lambda i: (0, 0)),
        ],
        out_specs=pl.BlockSpec((bm, D), lambda i: (i, 0)),
        out_shape=jax.ShapeDtypeStruct((N, D), jnp.float32),
    )(x, W1, b1.reshape(1, D), W2, b2.reshape(1, D))


@functools.partial(
    pl.kernel,
    mesh=_vector_mesh,
    out_type=jax.ShapeDtypeStruct((2, NC, NP, DH), jnp.bfloat16),
    scratch_types=[
        pltpu.VMEM((NCH, G), jnp.int32),
        pltpu.VMEM((NCH, G), jnp.int32),
        pltpu.VMEM((NCH, G), jnp.float32),
        pltpu.VMEM((G, DH), jnp.bfloat16),
        pltpu.VMEM((G, DH), jnp.bfloat16),
        pltpu.VMEM((G, DH), jnp.bfloat16),
        pltpu.VMEM((G, DH), jnp.bfloat16),
        pltpu.VMEM((G, DH), jnp.bfloat16),
        pltpu.VMEM((G, DH), jnp.bfloat16),
        pltpu.VMEM((G, DH), jnp.float32),
        pltpu.VMEM((G, DH), jnp.float32),
        pltpu.VMEM((RB, DH), jnp.bfloat16),
        pltpu.VMEM_SHARED((NP, DH), jnp.float32),
        pltpu.SemaphoreType.DMA,
        pltpu.SemaphoreType.DMA,
        pltpu.SemaphoreType.DMA,
        pltpu.SemaphoreType.DMA,
        pltpu.SemaphoreType.DMA,
        pltpu.SemaphoreType.DMA,
        pltpu.SemaphoreType.DMA,
        pltpu.SemaphoreType.DMA,
    ],
    compiler_params=_sc_params,
)
def _sc_prop(hs_hbm, row_hbm, col_hbm, val_hbm, st_hbm,
             rowv, colv, valv, gbuf0, gbuf1, gbuf2, gbuf3, gbuf4, gbuf5,
             sbuf0, sbuf1, obuf, agg,
             gsem0, gsem1, gsem2, gsem3, gsem4, gsem5, ssem0, ssem1):
    c = lax.axis_index("c")
    s = lax.axis_index("s")
    gbufs = (gbuf0, gbuf1, gbuf2, gbuf3, gbuf4, gbuf5)
    sbufs = (sbuf0, sbuf1)
    gsems = (gsem0, gsem1, gsem2, gsem3, gsem4, gsem5)
    ssems = (ssem0, ssem1)

    pltpu.sync_copy(row_hbm.at[s], rowv)
    pltpu.sync_copy(col_hbm.at[s], colv)
    pltpu.sync_copy(val_hbm.at[s], valv)

    pltpu.sync_copy(hs_hbm.at[c, pl.ds(s * RPT, RPT)],
                    st_hbm.at[0, c, pl.ds(s * RPT, RPT)])

    def _one_round(src, dst):
        h_src = st_hbm.at[src, c]

        @plsc.parallel_loop(0, RB, unroll=4)
        def _z(e):
            for j in range(0, DH, L):
                sbuf0[e, pl.ds(j, L)] = jnp.zeros((L,), jnp.float32)

        @pl.loop(0, RPT // RB)
        def _zc(rr):
            pltpu.sync_copy(sbuf0, agg.at[pl.ds(s * RPT + rr * RB, RB)])

        plsc.subcore_barrier()

        pltpu.async_copy(h_src.at[colv.at[0]], gbuf0, gsem0)
        pltpu.async_copy(h_src.at[colv.at[1]], gbuf1, gsem1)
        pltpu.async_copy(h_src.at[colv.at[2]], gbuf2, gsem2)
        pltpu.async_copy(h_src.at[colv.at[3]], gbuf3, gsem3)
        pltpu.async_copy(h_src.at[colv.at[4]], gbuf4, gsem4)
        pltpu.async_copy(h_src.at[colv.at[5]], gbuf5, gsem5)

        def _process(i, g4, b):
            @pl.when(i >= 2)
            def _():
                pltpu.make_async_copy(
                    sbufs[b], agg.at[rowv.at[i - 2]], ssems[b]).wait()

            pltpu.make_async_copy(
                h_src.at[colv.at[i]], gbufs[g4], gsems[g4]).wait()

            @plsc.parallel_loop(0, G, step=L, unroll=2)
            def _grp(e0):
                vv16 = valv[i, pl.ds(e0, L)]
                for k in range(L):
                    vv = vv16[k]
                    for j in range(0, DH, 2 * L):
                        hb = gbufs[g4][e0 + k, pl.ds(j, 2 * L)]
                        ha, hc = plsc.unpack(
                            hb, format=plsc.PackFormat.INTERLEAVED)
                        sbufs[b][e0 + k, pl.ds(j, L)] = ha * vv
                        sbufs[b][e0 + k, pl.ds(j + L, L)] = hc * vv

            pltpu.async_copy(sbufs[b], agg.at[rowv.at[i]], ssems[b],
                             add=True)

            @pl.when(i + 6 < NCH)
            def _():
                pltpu.async_copy(h_src.at[colv.at[i + 6]], gbufs[g4],
                                 gsems[g4])

        @pl.loop(0, NCH // 6)
        def _hex(t):
            for q in range(6):
                _process(6 * t + q, q, q % 2)

        for q in range(NCH - NCH % 6, NCH):
            _process(q, q % 6, q % 2)

        pltpu.make_async_copy(sbuf0, agg.at[rowv.at[NCH - 2]], ssem0).wait()
        pltpu.make_async_copy(sbuf1, agg.at[rowv.at[NCH - 1]], ssem1).wait()

        plsc.subcore_barrier()

        @pl.loop(0, RPT // RB)
        def _blend(rr):
            off = s * RPT + rr * RB
            pltpu.sync_copy(h_src.at[pl.ds(off, RB)], gbuf0)
            pltpu.sync_copy(agg.at[pl.ds(off, RB)], sbuf1)

            @plsc.parallel_loop(0, RB, unroll=2)
            def _b(e):
                for j in range(0, DH, 2 * L):
                    hb = gbuf0[e, pl.ds(j, 2 * L)]
                    ha, hc = plsc.unpack(
                        hb, format=plsc.PackFormat.INTERLEAVED)
                    ra = ((1.0 - ALPHA) * sbuf1[e, pl.ds(j, L)]
                          + ALPHA * ha)
                    rc = ((1.0 - ALPHA) * sbuf1[e, pl.ds(j + L, L)]
                          + ALPHA * hc)
                    obuf[e, pl.ds(j, 2 * L)] = plsc.pack(
                        ra, rc, format=plsc.PackFormat.INTERLEAVED)

            pltpu.sync_copy(obuf, st_hbm.at[dst, c, pl.ds(off, RB)])

    @pl.loop(0, K // 2)
    def _round_pair(r):
        _one_round(0, 1)
        _one_round(1, 0)


def kernel(x, adj_indices, adj_values, W1, b1, W2, b2):
    h = _mlp(x, W1, b1, W2, b2)
    hs = jnp.zeros((NC, NP, DH), jnp.bfloat16)
    hs = hs.at[:, :N, :].set(
        jnp.transpose(h.reshape(N, NC, DH), (1, 0, 2)).astype(jnp.bfloat16))
    row = adj_indices[0].reshape(NS, NCH, G)
    col = adj_indices[1].reshape(NS, NCH, G)
    vals = adj_values.reshape(NS, NCH, G)
    st = _sc_prop(hs, row, col, vals)
    return jnp.transpose(st[0, :, :N, :].astype(jnp.float32),
                         (1, 0, 2)).reshape(N, D)


# speedup vs baseline: 1.3461x; 1.3461x over previous
import dataclasses
import functools

import jax
import jax.numpy as jnp
from jax import lax
from jax.experimental import pallas as pl
from jax.experimental.pallas import tpu as pltpu
from jax.experimental.pallas import tpu_sc as plsc

ALPHA = 0.01
K = 10

N = 10000
E = 320000
D = 128

NC = 2
NS = 16
L = 16
DH = D // NC

NP = 10240
G = 80
NCH = 250
RPT = NP // NS
RB = 80

_vector_mesh = plsc.VectorSubcoreMesh(core_axis_name="c", subcore_axis_name="s")

_sc_params = pltpu.CompilerParams()
for _f, _v in (("needs_layout_passes", False), ("use_tc_tiling_on_sc", False)):
    if _f in pltpu.CompilerParams.__dataclass_fields__:
        _sc_params = dataclasses.replace(_sc_params, **{_f: _v})


def _mlp_body(x_ref, w1_ref, b1_ref, w2_ref, b2_ref, o_ref):
    h = lax.dot_general(x_ref[...], w1_ref[...], (((1,), (1,)), ((), ())),
                        preferred_element_type=jnp.float32,
                        precision=lax.Precision.HIGHEST)
    h = jnp.maximum(h + b1_ref[...], 0.0)
    o = lax.dot_general(h, w2_ref[...], (((1,), (1,)), ((), ())),
                        preferred_element_type=jnp.float32,
                        precision=lax.Precision.HIGHEST)
    o_ref[...] = o + b2_ref[...]


def _mlp(x, W1, b1, W2, b2):
    bm = 2000
    return pl.pallas_call(
        _mlp_body,
        grid=(N // bm,),
        in_specs=[
            pl.BlockSpec((bm, D), lambda i: (i, 0)),
            pl.BlockSpec((D, D), lambda i: (0, 0)),
            pl.BlockSpec((1, D), lambda i: (0, 0)),
            pl.BlockSpec((D, D), lambda i: (0, 0)),
            pl.BlockSpec((1, D), lambda i: (0, 0)),
        ],
        out_specs=pl.BlockSpec((bm, D), lambda i: (i, 0)),
        out_shape=jax.ShapeDtypeStruct((N, D), jnp.float32),
    )(x, W1, b1.reshape(1, D), W2, b2.reshape(1, D))


@functools.partial(
    pl.kernel,
    mesh=_vector_mesh,
    out_type=jax.ShapeDtypeStruct((2, NC, NP, DH), jnp.bfloat16),
    scratch_types=[
        pltpu.VMEM((NCH, G), jnp.int32),
        pltpu.VMEM((NCH, G), jnp.int32),
        pltpu.VMEM((NCH, G), jnp.float32),
        pltpu.VMEM((G, DH), jnp.bfloat16),
        pltpu.VMEM((G, DH), jnp.bfloat16),
        pltpu.VMEM((G, DH), jnp.bfloat16),
        pltpu.VMEM((G, DH), jnp.bfloat16),
        pltpu.VMEM((G, DH), jnp.float32),
        pltpu.VMEM((G, DH), jnp.float32),
        pltpu.VMEM((RB, DH), jnp.float32),
        pltpu.VMEM((RB, DH), jnp.bfloat16),
        pltpu.VMEM_SHARED((NP, DH), jnp.float32),
        pltpu.SemaphoreType.DMA,
        pltpu.SemaphoreType.DMA,
        pltpu.SemaphoreType.DMA,
        pltpu.SemaphoreType.DMA,
        pltpu.SemaphoreType.DMA,
        pltpu.SemaphoreType.DMA,
    ],
    compiler_params=_sc_params,
)
def _sc_prop(hs_hbm, row_hbm, col_hbm, val_hbm, st_hbm,
             rowv, colv, valv, gbuf0, gbuf1, gbuf2, gbuf3,
             sbuf0, sbuf1, zbuf, obuf, agg,
             gsem0, gsem1, gsem2, gsem3, ssem0, ssem1):
    c = lax.axis_index("c")
    s = lax.axis_index("s")
    gbufs = (gbuf0, gbuf1, gbuf2, gbuf3)
    sbufs = (sbuf0, sbuf1)
    gsems = (gsem0, gsem1, gsem2, gsem3)
    ssems = (ssem0, ssem1)

    pltpu.sync_copy(row_hbm.at[s], rowv)
    pltpu.sync_copy(col_hbm.at[s], colv)
    pltpu.sync_copy(val_hbm.at[s], valv)

    @plsc.parallel_loop(0, RB, unroll=4)
    def _z(e):
        for j in range(0, DH, L):
            zbuf[e, pl.ds(j, L)] = jnp.zeros((L,), jnp.float32)

    pltpu.sync_copy(hs_hbm.at[c, pl.ds(s * RPT, RPT)],
                    st_hbm.at[0, c, pl.ds(s * RPT, RPT)])

    def _one_round(src, dst):
        h_src = st_hbm.at[src, c]

        @pl.loop(0, RPT // RB)
        def _zc(rr):
            pltpu.sync_copy(zbuf, agg.at[pl.ds(s * RPT + rr * RB, RB)])

        plsc.subcore_barrier()

        pltpu.async_copy(h_src.at[colv.at[0]], gbuf0, gsem0)
        pltpu.async_copy(h_src.at[colv.at[1]], gbuf1, gsem1)
        pltpu.async_copy(h_src.at[colv.at[2]], gbuf2, gsem2)
        pltpu.async_copy(h_src.at[colv.at[3]], gbuf3, gsem3)

        def _process(i, g4, b):
            @pl.when(i >= 2)
            def _():
                pltpu.make_async_copy(
                    sbufs[b], agg.at[rowv.at[i - 2]], ssems[b]).wait()

            pltpu.make_async_copy(
                h_src.at[colv.at[i]], gbufs[g4], gsems[g4]).wait()

            @plsc.parallel_loop(0, G, step=L, unroll=2)
            def _grp(e0):
                vv16 = valv[i, pl.ds(e0, L)]
                for k in range(L):
                    vv = vv16[k]
                    for j in range(0, DH, 2 * L):
                        hb = gbufs[g4][e0 + k, pl.ds(j, 2 * L)]
                        ha, hc = plsc.unpack(
                            hb, format=plsc.PackFormat.INTERLEAVED)
                        sbufs[b][e0 + k, pl.ds(j, L)] = ha * vv
                        sbufs[b][e0 + k, pl.ds(j + L, L)] = hc * vv

            pltpu.async_copy(sbufs[b], agg.at[rowv.at[i]], ssems[b],
                             add=True)

            @pl.when(i + 4 < NCH)
            def _():
                pltpu.async_copy(h_src.at[colv.at[i + 4]], gbufs[g4],
                                 gsems[g4])

        @pl.loop(0, NCH // 4)
        def _quad(t):
            for q in range(4):
                _process(4 * t + q, q, q % 2)

        for q in range(NCH - NCH % 4, NCH):
            _process(q, q % 4, q % 2)

        pltpu.make_async_copy(sbuf0, agg.at[rowv.at[NCH - 2]], ssem0).wait()
        pltpu.make_async_copy(sbuf1, agg.at[rowv.at[NCH - 1]], ssem1).wait()

        plsc.subcore_barrier()

        @pl.loop(0, RPT // RB)
        def _blend(rr):
            off = s * RPT + rr * RB
            pltpu.sync_copy(h_src.at[pl.ds(off, RB)], gbuf0)
            pltpu.sync_copy(agg.at[pl.ds(off, RB)], sbuf1)

            @plsc.parallel_loop(0, RB, unroll=2)
            def _b(e):
                for j in range(0, DH, 2 * L):
                    hb = gbuf0[e, pl.ds(j, 2 * L)]
                    ha, hc = plsc.unpack(
                        hb, format=plsc.PackFormat.INTERLEAVED)
                    ra = ((1.0 - ALPHA) * sbuf1[e, pl.ds(j, L)]
                          + ALPHA * ha)
                    rc = ((1.0 - ALPHA) * sbuf1[e, pl.ds(j + L, L)]
                          + ALPHA * hc)
                    obuf[e, pl.ds(j, 2 * L)] = plsc.pack(
                        ra, rc, format=plsc.PackFormat.INTERLEAVED)

            pltpu.sync_copy(obuf, st_hbm.at[dst, c, pl.ds(off, RB)])

    @pl.loop(0, K // 2)
    def _round_pair(r):
        _one_round(0, 1)
        _one_round(1, 0)


def kernel(x, adj_indices, adj_values, W1, b1, W2, b2):
    h = _mlp(x, W1, b1, W2, b2)
    hs = jnp.zeros((NC, NP, DH), jnp.bfloat16)
    hs = hs.at[:, :N, :].set(
        jnp.transpose(h.reshape(N, NC, DH), (1, 0, 2)).astype(jnp.bfloat16))
    row = adj_indices[0].reshape(NS, NCH, G)
    col = adj_indices[1].reshape(NS, NCH, G)
    vals = adj_values.reshape(NS, NCH, G)
    st = _sc_prop(hs, row, col, vals)
    return jnp.transpose(st[0, :, :N, :].astype(jnp.float32),
                         (1, 0, 2)).reshape(N, D)
